# Initial kernel scaffold; baseline (speedup 1.0000x reference)
import jax
import jax.numpy as jnp
from jax.experimental import pallas as pl


def kernel(W1, b1, W2, b2, W3, b3):
    raise NotImplementedError("write your pallas kernel here")


# speedup vs baseline: 2.0732x; 2.0732x over previous
import functools

import jax
import jax.numpy as jnp
from jax.experimental import pallas as pl

_H = 512
_BLK = 1024
_RES = [65, 129, 257, 513]
_NPT = [0, 8192, 16384, 32768]


def _mlp_body(c_ref, w1_ref, b1_ref, w2_ref, b2_ref, w3_ref, b3_ref, out_ref):
    c = c_ref[...]
    x = c[:, 0:1]
    y = c[:, 1:2]
    h = jnp.tanh(x * w1_ref[0:1, :] + y * w1_ref[1:2, :] + b1_ref[...])
    h = jnp.tanh(
        jnp.dot(h, w2_ref[...], preferred_element_type=jnp.float32) + b2_ref[...]
    )
    z = jnp.sum(h * w3_ref[...], axis=1, keepdims=True) + b3_ref[...]
    out_ref[...] = jax.nn.sigmoid(z)


def _mlp(c, W1, b1r, W2, b2r, w3r, b3r):
    n = c.shape[0]
    rep = lambda i: (0, 0)
    out = pl.pallas_call(
        _mlp_body,
        grid=(n // _BLK,),
        in_specs=[
            pl.BlockSpec((_BLK, 2), lambda i: (i, 0)),
            pl.BlockSpec((2, _H), rep),
            pl.BlockSpec((1, _H), rep),
            pl.BlockSpec((_H, _H), rep),
            pl.BlockSpec((1, _H), rep),
            pl.BlockSpec((1, _H), rep),
            pl.BlockSpec((1, 1), rep),
        ],
        out_specs=pl.BlockSpec((_BLK, 1), lambda i: (i, 0)),
        out_shape=jax.ShapeDtypeStruct((n, 1), jnp.float32),
    )(c, W1, b1r, W2, b2r, w3r, b3r)
    return out[:, 0]


def _up2(o):
    r = o.shape[0]
    big = 2 * r - 1
    out = jnp.zeros((big, big), o.dtype)
    out = out.at[::2, ::2].set(o)
    out = out.at[::2, 1::2].set(o[:, :-1] * 0.5 + o[:, 1:] * 0.5)
    out = out.at[1::2, ::2].set(o[:-1, :] * 0.5 + o[1:, :] * 0.5)
    out = out.at[1::2, 1::2].set(
        o[:-1, :-1] * 0.25 + o[:-1, 1:] * 0.25 + o[1:, :-1] * 0.25 + o[1:, 1:] * 0.25
    )
    return out


def _pad_pts(c, mult):
    n = c.shape[0]
    npad = (-n) % mult
    if npad:
        c = jnp.concatenate([c, jnp.zeros((npad, 2), c.dtype)], axis=0)
    return c, n


def kernel(W1, b1, W2, b2, W3, b3):
    b1r = b1.reshape(1, _H)
    b2r = b2.reshape(1, _H)
    w3r = W3.reshape(1, _H)
    b3r = b3.reshape(1, 1)
    mlp = lambda c: _mlp(c, W1, b1r, W2, b2r, w3r, b3r)

    ax = jnp.linspace(0.0, 512.0, _RES[0]).astype(jnp.int32).astype(jnp.float32)
    gx = jnp.tile(ax, _RES[0])
    gy = jnp.repeat(ax, _RES[0])
    c0 = jnp.stack([gx, gy], axis=-1) / 512.0 * 2.0 - 1.0
    c0p, n0 = _pad_pts(c0, _BLK)
    occ = mlp(c0p)[:n0].reshape(_RES[0], _RES[0])

    for res, npt in zip(_RES[1:], _NPT[1:]):
        stride = (_RES[-1] - 1) / (res - 1)
        occ = _up2(occ)
        unc = -jnp.abs(occ - 0.5).reshape(-1)
        _, idx = jax.lax.top_k(unc, npt)
        px = (idx % res).astype(jnp.float32)
        py = (idx // res).astype(jnp.float32)
        c = jnp.stack([px, py], axis=-1) * stride / 512.0 * 2.0 - 1.0
        vals = mlp(c)
        occ = occ.reshape(-1).at[idx].set(vals).reshape(res, res)

    out = occ[None, None]
    return jnp.broadcast_to(out, (4, 1, _RES[-1], _RES[-1]))


# speedup vs baseline: 4.7186x; 2.2760x over previous
import functools

import jax
import jax.numpy as jnp
from jax.experimental import pallas as pl

_H = 512
_BLK = 1024
_RES = [65, 129, 257, 513]
_NPT = [0, 8192, 16384, 32768]


def _mlp_body(c_ref, w1_ref, b1_ref, w2_ref, b2_ref, w3_ref, b3_ref, out_ref):
    c = c_ref[...]
    x = c[:, 0:1]
    y = c[:, 1:2]
    h = jnp.tanh(x * w1_ref[0:1, :] + y * w1_ref[1:2, :] + b1_ref[...])
    h = jnp.tanh(
        jnp.dot(h, w2_ref[...], preferred_element_type=jnp.float32) + b2_ref[...]
    )
    z = jnp.sum(h * w3_ref[...], axis=1, keepdims=True) + b3_ref[...]
    out_ref[...] = jax.nn.sigmoid(z)


def _mlp(c, W1, b1r, W2, b2r, w3r, b3r):
    n = c.shape[0]
    rep = lambda i: (0, 0)
    out = pl.pallas_call(
        _mlp_body,
        grid=(n // _BLK,),
        in_specs=[
            pl.BlockSpec((_BLK, 2), lambda i: (i, 0)),
            pl.BlockSpec((2, _H), rep),
            pl.BlockSpec((1, _H), rep),
            pl.BlockSpec((_H, _H), rep),
            pl.BlockSpec((1, _H), rep),
            pl.BlockSpec((1, _H), rep),
            pl.BlockSpec((1, 1), rep),
        ],
        out_specs=pl.BlockSpec((_BLK, 1), lambda i: (i, 0)),
        out_shape=jax.ShapeDtypeStruct((n, 1), jnp.float32),
    )(c, W1, b1r, W2, b2r, w3r, b3r)
    return out[:, 0]


def _interp_matrix(r):
    big = 2 * r - 1
    u = jnp.zeros((big, r), jnp.float32)
    i = jnp.arange(r)
    u = u.at[2 * i, i].set(1.0)
    j = jnp.arange(r - 1)
    u = u.at[2 * j + 1, j].set(0.5)
    u = u.at[2 * j + 1, j + 1].set(0.5)
    return u


def _up_body(occ_ref, u_ref, ut_ref, out_ref):
    tmp = jnp.dot(u_ref[...], occ_ref[...], preferred_element_type=jnp.float32)
    out_ref[...] = jnp.dot(tmp, ut_ref[...], preferred_element_type=jnp.float32)


def _up2(o):
    r = o.shape[0]
    big = 2 * r - 1
    u = _interp_matrix(r)
    full = lambda a: pl.BlockSpec(a.shape, lambda: (0,) * a.ndim)
    return pl.pallas_call(
        _up_body,
        in_specs=[full(o), full(u), full(u.T)],
        out_specs=pl.BlockSpec((big, big), lambda: (0, 0)),
        out_shape=jax.ShapeDtypeStruct((big, big), jnp.float32),
    )(o, u, u.T)


def _pad_pts(c, mult):
    n = c.shape[0]
    npad = (-n) % mult
    if npad:
        c = jnp.concatenate([c, jnp.zeros((npad, 2), c.dtype)], axis=0)
    return c, n


def kernel(W1, b1, W2, b2, W3, b3):
    b1r = b1.reshape(1, _H)
    b2r = b2.reshape(1, _H)
    w3r = W3.reshape(1, _H)
    b3r = b3.reshape(1, 1)
    mlp = lambda c: _mlp(c, W1, b1r, W2, b2r, w3r, b3r)

    ax = jnp.linspace(0.0, 512.0, _RES[0]).astype(jnp.int32).astype(jnp.float32)
    gx = jnp.tile(ax, _RES[0])
    gy = jnp.repeat(ax, _RES[0])
    c0 = jnp.stack([gx, gy], axis=-1) / 512.0 * 2.0 - 1.0
    c0p, n0 = _pad_pts(c0, _BLK)
    occ = mlp(c0p)[:n0].reshape(_RES[0], _RES[0])

    for res, npt in zip(_RES[1:], _NPT[1:]):
        stride = (_RES[-1] - 1) / (res - 1)
        occ = _up2(occ)
        unc = -jnp.abs(occ - 0.5).reshape(-1)
        _, idx = jax.lax.top_k(unc, npt)
        px = (idx % res).astype(jnp.float32)
        py = (idx // res).astype(jnp.float32)
        c = jnp.stack([px, py], axis=-1) * stride / 512.0 * 2.0 - 1.0
        vals = mlp(c)
        occ = occ.reshape(-1).at[idx].set(vals).reshape(res, res)

    out = occ[None, None]
    return jnp.broadcast_to(out, (4, 1, _RES[-1], _RES[-1]))
